# Initial kernel scaffold; baseline (speedup 1.0000x reference)
import jax
import jax.numpy as jnp
from jax.experimental import pallas as pl


def kernel(x, pos, W_lin, W_src, W_dst, W_pos, b_pos, W_mlp, b_mlp):
    raise NotImplementedError("write your pallas kernel here")


# speedup vs baseline: 3.9071x; 3.9071x over previous
import functools

import jax
import jax.numpy as jnp
from jax import lax
from jax.experimental import pallas as pl
from jax.experimental.pallas import tpu as pltpu
from jax.experimental.pallas import tpu_sc as plsc

_N = 10000
_D = 128
_K = 32
_BR = 128
_NBLK = 79
_NPAD = _NBLK * _BR
_DIN = 136

_F32 = jnp.float32
_I32 = jnp.int32

_SC_NC = 2
_SC_NS = 16
_SC_NW = _SC_NC * _SC_NS
_EDGES = _N * _K
_E_PER_W = _EDGES // _SC_NW
_GCHUNK = 80
_GITERS = _E_PER_W // _GCHUNK


def _stageA_body(xp_ref, wc_ref, bvec_ref, g_ref, cq_ref, xmax_ref):
    i = pl.program_id(0)
    xb = xp_ref[...]
    y = lax.dot_general(xb, wc_ref[...], (((1,), (0,)), ((), ())),
                        precision=lax.Precision.HIGHEST)
    y = y + bvec_ref[...]
    g_ref[...] = y[:, :256]
    cq_ref[...] = y[:, 256:512]
    row = i * _BR + lax.broadcasted_iota(_I32, (_BR, 1), 0)
    xm = jnp.where(row < _N, xb[:, :_D], -jnp.inf)
    bmax = jnp.max(xm, axis=0, keepdims=True)

    @pl.when(i == 0)
    def _():
        xmax_ref[...] = jnp.full((1, _D), -jnp.inf, _F32)

    xmax_ref[...] = jnp.maximum(xmax_ref[...], bmax)


def _stageA(xp_pad, wc, bvec):
    return pl.pallas_call(
        _stageA_body,
        grid=(_NBLK,),
        in_specs=[
            pl.BlockSpec((_BR, _DIN), lambda i: (i, 0)),
            pl.BlockSpec((_DIN, 512), lambda i: (0, 0)),
            pl.BlockSpec((1, 512), lambda i: (0, 0)),
        ],
        out_specs=[
            pl.BlockSpec((_BR, 256), lambda i: (i, 0)),
            pl.BlockSpec((_BR, 256), lambda i: (i, 0)),
            pl.BlockSpec((1, _D), lambda i: (0, 0)),
        ],
        out_shape=[
            jax.ShapeDtypeStruct((_N, 256), _F32),
            jax.ShapeDtypeStruct((_N, 256), _F32),
            jax.ShapeDtypeStruct((1, _D), _F32),
        ],
    )(xp_pad, wc, bvec)


def _stageB_body(xr_ref, xt_ref, idx_ref, dist_ref):
    i = pl.program_id(0)
    xr = xr_ref[...]
    xt = xt_ref[...]
    sqi = jnp.sum(xr * xr, axis=1, keepdims=True)
    sqj = jnp.sum(xt * xt, axis=0, keepdims=True)
    mm = lax.dot_general(xr, xt, (((1,), (0,)), ((), ())))
    col = lax.broadcasted_iota(_I32, (_BR, _NPAD), 1)
    row = i * _BR + lax.broadcasted_iota(_I32, (_BR, _NPAD), 0)
    dist = (sqi + sqj) - 2.0 * mm
    dist = dist + jnp.where(col == row, _F32(1e10), _F32(0.0))
    dist = jnp.where(col >= _N, jnp.inf, dist)
    dist_ref[...] = dist

    lv = jnp.full((_BR, 1), -jnp.inf, _F32)
    li = jnp.full((_BR, 1), -1, _I32)
    big = _I32(2**30)
    for k in range(_K):
        d = dist_ref[...]
        valid = (d > lv) | ((d == lv) & (col > li))
        cand = jnp.where(valid, d, jnp.inf)
        m = jnp.min(cand, axis=1, keepdims=True)
        j = jnp.min(jnp.where(cand == m, col, big), axis=1, keepdims=True)
        idx_ref[:, pl.ds(k, 1)] = j
        lv, li = m, j


def _stageB(xpad, xt):
    return pl.pallas_call(
        _stageB_body,
        grid=(_NBLK,),
        in_specs=[
            pl.BlockSpec((_BR, _D), lambda i: (i, 0)),
            pl.BlockSpec((_D, _NPAD), lambda i: (0, 0)),
        ],
        out_specs=pl.BlockSpec((_BR, _K), lambda i: (i, 0)),
        out_shape=jax.ShapeDtypeStruct((_NPAD, _K), _I32),
        scratch_shapes=[pltpu.VMEM((_BR, _NPAD), _F32)],
    )(xpad, xt)


def _stageC(table, idx):
    mesh = plsc.VectorSubcoreMesh(core_axis_name="c", subcore_axis_name="s")

    @functools.partial(
        pl.kernel,
        mesh=mesh,
        out_type=jax.ShapeDtypeStruct((_EDGES, 256), _F32),
        scratch_types=[
            pltpu.VMEM((_E_PER_W,), _I32),
            pltpu.VMEM((_GCHUNK, 256), _F32),
            pltpu.SemaphoreType.DMA,
        ],
    )
    def k(table_hbm, idx_hbm, out_hbm, idx_v, rows_v, sem):
        wid = lax.axis_index("s") * _SC_NC + lax.axis_index("c")
        base = wid * _E_PER_W
        pltpu.sync_copy(idx_hbm.at[pl.ds(base, _E_PER_W)], idx_v)

        def body(t, _):
            off = pl.multiple_of(t * _GCHUNK, 8)
            pltpu.async_copy(
                table_hbm.at[idx_v.at[pl.ds(off, _GCHUNK)]], rows_v, sem
            ).wait()
            pltpu.sync_copy(rows_v, out_hbm.at[pl.ds(base + off, _GCHUNK)])
            return ()

        lax.fori_loop(0, _GITERS, body, (), unroll=False)

    return k(table, idx)


def _stageD_body(gath_ref, cq_ref, g_ref, xmax_ref, w1_ref, w2_ref, bm_ref,
                 out_ref):
    cq = cq_ref[...]
    g = g_ref[...]
    c = cq[:, :_D]
    q = cq[:, _D:]
    s_own = g[:, :_D]
    u_own = g[:, _D:]

    a_self = c - s_own
    m = a_self
    for k in range(_K):
        m = jnp.maximum(m, c - gath_ref[:, k * 256:k * 256 + _D])
    es = jnp.exp(a_self - m)
    den = es
    num = es * u_own
    for k in range(_K):
        blk = gath_ref[:, pl.ds(k * 256, 256)]
        e = jnp.exp((c - blk[:, :_D]) - m)
        den = den + e
        num = num + e * blk[:, _D:]
    h1 = (num + den * q) / (den + _F32(1e-16))

    r = lax.dot_general(xmax_ref[...], w2_ref[...], (((1,), (0,)), ((), ())),
                        precision=lax.Precision.HIGHEST) + bm_ref[...]
    o = lax.dot_general(h1, w1_ref[...], (((1,), (0,)), ((), ())),
                        precision=lax.Precision.HIGHEST) + r
    out_ref[...] = jnp.maximum(o, 0.0)


def _stageD(gath2, cq, g, xmax, w1, w2, bm):
    return pl.pallas_call(
        _stageD_body,
        grid=(_NBLK,),
        in_specs=[
            pl.BlockSpec((_BR, _K * 256), lambda i: (i, 0)),
            pl.BlockSpec((_BR, 256), lambda i: (i, 0)),
            pl.BlockSpec((_BR, 256), lambda i: (i, 0)),
            pl.BlockSpec((1, _D), lambda i: (0, 0)),
            pl.BlockSpec((_D, _D), lambda i: (0, 0)),
            pl.BlockSpec((_D, _D), lambda i: (0, 0)),
            pl.BlockSpec((1, _D), lambda i: (0, 0)),
        ],
        out_specs=pl.BlockSpec((_BR, _D), lambda i: (i, 0)),
        out_shape=jax.ShapeDtypeStruct((_N, _D), _F32),
    )(gath2, cq, g, xmax, w1, w2, bm)


def kernel(x, pos, W_lin, W_src, W_dst, W_pos, b_pos, W_mlp, b_mlp):
    xpad = jnp.zeros((_NPAD, _D), _F32).at[:_N].set(x)
    xt = xpad.T

    pz = jnp.zeros((_DIN - _D - 3, 512), _F32)
    wx = jnp.concatenate([W_src, W_lin, W_dst, jnp.zeros((_D, _D), _F32)],
                         axis=1)
    wp = jnp.concatenate([W_pos, -W_pos, W_pos, W_pos], axis=1)
    wc = jnp.concatenate([wx, wp, pz], axis=0)
    zb = jnp.zeros((256,), _F32)
    bvec = jnp.concatenate([zb, b_pos, b_pos]).reshape(1, 512)

    xp_pad = jnp.zeros((_NPAD, _DIN), _F32)
    xp_pad = xp_pad.at[:_N, :_D].set(x).at[:_N, _D:_D + 3].set(pos)

    g, cq, xmax = _stageA(xp_pad, wc, bvec)
    idx = _stageB(xpad, xt)[:_N, :_K].reshape(-1)
    gath = _stageC(g, idx)
    gath2 = gath.reshape(_N, _K * 256)

    w1 = W_mlp[:_D, :]
    w2 = W_mlp[_D:, :]
    bm = b_mlp.reshape(1, _D)
    return _stageD(gath2, cq, g, xmax, w1, w2, bm)
